# baseline (speedup 1.0000x reference)
import functools

import jax
import jax.numpy as jnp
import numpy as np
from jax import lax
from jax.experimental import pallas as pl
from jax.experimental.pallas import tpu as pltpu
from jax.experimental.pallas import tpu_sc as plsc

N = 10000
D = 128
E = 320000
NC = 2
NS = 16
CH = 128
E_PAD = 327680
N_SP = 10112
ZROWS = N_SP // NS
OROWS = N_SP // NS


NQ = 8
K0 = 160
QC0 = K0 // NQ


def _sc_body(r_hbm, idx0_hbm, out_hbm,
             i0, i1, rows0, rows1, agg_sh, sem_i, sem_r0, sem_r1):
    c = lax.axis_index("c")
    s = lax.axis_index("s")

    def init():
        def zrow(i, carry):
            for j in range(8):
                rows0[i, pl.ds(16 * j, 16)] = jnp.zeros((16,), jnp.float32)
            return carry

        lax.fori_loop(0, CH, zrow, 0)
        zbase = s * ZROWS
        for k in range(ZROWS // CH):
            pltpu.sync_copy(rows0, agg_sh.at[pl.ds(zbase + k * CH, CH)])
        rem = ZROWS % CH
        pltpu.sync_copy(rows0.at[pl.ds(0, rem)],
                        agg_sh.at[pl.ds(zbase + ZROWS - rem, rem)])

    def run(idx_hbm, qch):
        pltpu.sync_copy(idx_hbm.at[s, 0], i0.at[pl.ds(0, 2 * qch)])
        plsc.subcore_barrier()

        def gather(iq, k, rows, sem):
            pltpu.async_copy(r_hbm.at[iq.at[k]], rows, sem)

        def gwait(iq, k, rows, sem):
            pltpu.make_async_copy(r_hbm.at[iq.at[k]], rows, sem).wait()

        def scat(iq, k, rows):
            pltpu.sync_copy(rows, agg_sh.at[iq.at[qch + k]], add=True)

        for q in range(NQ):
            iq, inx = (i0, i1) if q % 2 == 0 else (i1, i0)
            if q < NQ - 1:
                pltpu.async_copy(idx_hbm.at[s, q + 1],
                                 inx.at[pl.ds(0, 2 * qch)], sem_i)
            gather(iq, 0, rows0, sem_r0)

            def body(j, carry):
                gather(iq, 2 * j + 1, rows1, sem_r1)
                gwait(iq, 2 * j, rows0, sem_r0)
                scat(iq, 2 * j, rows0)
                gather(iq, 2 * j + 2, rows0, sem_r0)
                gwait(iq, 2 * j + 1, rows1, sem_r1)
                scat(iq, 2 * j + 1, rows1)
                return carry

            lax.fori_loop(0, qch // 2 - 1, body, 0)
            gather(iq, qch - 1, rows1, sem_r1)
            gwait(iq, qch - 2, rows0, sem_r0)
            scat(iq, qch - 2, rows0)
            gwait(iq, qch - 1, rows1, sem_r1)
            scat(iq, qch - 1, rows1)
            if q < NQ - 1:
                pltpu.make_async_copy(idx_hbm.at[s, q + 1],
                                      inx.at[pl.ds(0, 2 * qch)], sem_i).wait()

    @pl.when(c == 0)
    def _():
        init()
        run(idx0_hbm, QC0)
        plsc.subcore_barrier()
        pltpu.sync_copy(agg_sh.at[pl.ds(s * OROWS, OROWS)],
                        out_hbm.at[pl.ds(s * OROWS, OROWS)])


@functools.cache
def _make_sc_scatter():
    return pl.kernel(
        _sc_body,
        out_type=jax.ShapeDtypeStruct((N_SP, D), jnp.float32),
        mesh=plsc.VectorSubcoreMesh(core_axis_name="c", subcore_axis_name="s",
                                    num_cores=NC, num_subcores=NS),
        scratch_types=[
            pltpu.VMEM((2 * QC0, CH), jnp.int32),
            pltpu.VMEM((2 * QC0, CH), jnp.int32),
            pltpu.VMEM((CH, D), jnp.float32),
            pltpu.VMEM((CH, D), jnp.float32),
            pltpu.VMEM_SHARED((N_SP, D), jnp.float32),
            pltpu.SemaphoreType.DMA,
            pltpu.SemaphoreType.DMA,
            pltpu.SemaphoreType.DMA,
        ],
    )


def _sc_scatter(r, idx0):
    return _make_sc_scatter()(r, idx0)


R = 1000
GRID = N // R

_CBN = float(np.float32(1.0) / np.sqrt(np.float32(1.0 + 1e-5)))


def _dot(a, b):
    return jnp.dot(a, b, preferred_element_type=jnp.float32)


def _kin_body(x_ref, w_ref, b_ref, vn_ref, hvn_ref, r_ref, pooled_ref):
    h0 = _dot(x_ref[...], w_ref[...]) + b_ref[...]
    hvn = h0 + vn_ref[...]
    hvn_ref[...] = hvn
    r_ref[...] = jnp.maximum(hvn, 0.0)
    ps = jnp.sum(hvn, axis=0, keepdims=True)

    @pl.when(pl.program_id(0) == 0)
    def _():
        pooled_ref[...] = ps

    @pl.when(pl.program_id(0) != 0)
    def _():
        pooled_ref[...] += ps


def _k_in(x, w_in, b_in, vn0):
    return pl.pallas_call(
        _kin_body,
        grid=(GRID,),
        in_specs=[
            pl.BlockSpec((R, D), lambda i: (i, 0)),
            pl.BlockSpec((D, D), lambda i: (0, 0)),
            pl.BlockSpec((1, D), lambda i: (0, 0)),
            pl.BlockSpec((1, D), lambda i: (0, 0)),
        ],
        out_specs=[
            pl.BlockSpec((R, D), lambda i: (i, 0)),
            pl.BlockSpec((R, D), lambda i: (i, 0)),
            pl.BlockSpec((1, D), lambda i: (0, 0)),
        ],
        out_shape=[
            jax.ShapeDtypeStruct((N, D), jnp.float32),
            jax.ShapeDtypeStruct((N, D), jnp.float32),
            jax.ShapeDtypeStruct((1, D), jnp.float32),
        ],
    )(x, w_in, b_in, vn0)


def _vnm_body(pooled_ref, vn_ref, w1_ref, b1_ref, g1_ref, be1_ref,
              w2_ref, b2_ref, g2_ref, be2_ref, out_ref):
    t = pooled_ref[...] + vn_ref[...]
    t = _dot(t, w1_ref[...]) + b1_ref[...]
    t = t * _CBN * g1_ref[...] + be1_ref[...]
    t = jnp.maximum(t, 0.0)
    t = _dot(t, w2_ref[...]) + b2_ref[...]
    t = t * _CBN * g2_ref[...] + be2_ref[...]
    out_ref[...] = jnp.maximum(t, 0.0)


def _k_vnm(pooled, vn, w1, b1, g1, be1, w2, b2, g2, be2):
    r2 = lambda a: a.reshape(1, D)
    return pl.pallas_call(
        _vnm_body,
        out_shape=jax.ShapeDtypeStruct((1, D), jnp.float32),
    )(pooled, vn, w1, r2(b1), r2(g1), r2(be1), w2, r2(b2), r2(g2), r2(be2))


def _gin_mlp(zsum, w1_ref, b1_ref, g1_ref, be1_ref, w2_ref, b2_ref,
             g2_ref, be2_ref):
    z = _dot(zsum, w1_ref[...]) + b1_ref[...]
    z = z * _CBN * g1_ref[...] + be1_ref[...]
    z = jnp.maximum(z, 0.0)
    z = _dot(z, w2_ref[...]) + b2_ref[...]
    z = z * _CBN * g2_ref[...] + be2_ref[...]
    return jnp.maximum(z, 0.0)


_W_SPECS = [
    pl.BlockSpec((D, D), lambda i: (0, 0)),
    pl.BlockSpec((1, D), lambda i: (0, 0)),
    pl.BlockSpec((1, D), lambda i: (0, 0)),
    pl.BlockSpec((1, D), lambda i: (0, 0)),
    pl.BlockSpec((D, D), lambda i: (0, 0)),
    pl.BlockSpec((1, D), lambda i: (0, 0)),
    pl.BlockSpec((1, D), lambda i: (0, 0)),
    pl.BlockSpec((1, D), lambda i: (0, 0)),
]


def _kmid_body(eps_ref, hvn_ref, p0_ref, w1_ref, b1_ref, g1_ref,
               be1_ref, w2_ref, b2_ref, g2_ref, be2_ref, vnn_ref,
               hvn1_ref, r1_ref, nrep_ref, pooled_ref):
    hvn = hvn_ref[...]
    zsum = (1.0 + eps_ref[0]) * hvn + p0_ref[...]
    out = _gin_mlp(zsum, w1_ref, b1_ref, g1_ref, be1_ref,
                   w2_ref, b2_ref, g2_ref, be2_ref)
    hvn1 = out + vnn_ref[...]
    hvn1_ref[...] = hvn1
    r1_ref[...] = jnp.maximum(hvn1, 0.0)
    nrep_ref[...] = hvn + hvn1
    ps = jnp.sum(hvn1, axis=0, keepdims=True)

    @pl.when(pl.program_id(0) == 0)
    def _():
        pooled_ref[...] = ps

    @pl.when(pl.program_id(0) != 0)
    def _():
        pooled_ref[...] += ps


def _k_mid(eps, hvn, p, weights, vnn):
    return pl.pallas_call(
        _kmid_body,
        grid=(GRID,),
        in_specs=[
            pl.BlockSpec(memory_space=pltpu.SMEM),
            pl.BlockSpec((R, D), lambda i: (i, 0)),
            pl.BlockSpec((R, D), lambda i: (i, 0)),
            *_W_SPECS,
            pl.BlockSpec((1, D), lambda i: (0, 0)),
        ],
        out_specs=[
            pl.BlockSpec((R, D), lambda i: (i, 0)),
            pl.BlockSpec((R, D), lambda i: (i, 0)),
            pl.BlockSpec((R, D), lambda i: (i, 0)),
            pl.BlockSpec((1, D), lambda i: (0, 0)),
        ],
        out_shape=[
            jax.ShapeDtypeStruct((N, D), jnp.float32),
            jax.ShapeDtypeStruct((N, D), jnp.float32),
            jax.ShapeDtypeStruct((N, D), jnp.float32),
            jax.ShapeDtypeStruct((1, D), jnp.float32),
        ],
    )(eps, hvn, p, *weights, vnn)


def _kfin_body(eps_ref, hvn_ref, p0_ref, w1_ref, b1_ref, g1_ref,
               be1_ref, w2_ref, b2_ref, g2_ref, be2_ref, vnn_ref, nrep_ref,
               out_ref):
    hvn = hvn_ref[...]
    zsum = (1.0 + eps_ref[0]) * hvn + p0_ref[...]
    out = _gin_mlp(zsum, w1_ref, b1_ref, g1_ref, be1_ref,
                   w2_ref, b2_ref, g2_ref, be2_ref)
    out_ref[...] = nrep_ref[...] + (out + vnn_ref[...])


def _k_fin(eps, hvn, p, weights, vnn, nrep):
    return pl.pallas_call(
        _kfin_body,
        grid=(GRID,),
        in_specs=[
            pl.BlockSpec(memory_space=pltpu.SMEM),
            pl.BlockSpec((R, D), lambda i: (i, 0)),
            pl.BlockSpec((R, D), lambda i: (i, 0)),
            *_W_SPECS,
            pl.BlockSpec((1, D), lambda i: (0, 0)),
            pl.BlockSpec((R, D), lambda i: (i, 0)),
        ],
        out_specs=pl.BlockSpec((R, D), lambda i: (i, 0)),
        out_shape=jax.ShapeDtypeStruct((N, D), jnp.float32),
    )(eps, hvn, p, *weights, vnn, nrep)


def kernel(x, edge_index, W_in, b_in, vn_emb, gin_eps, gin_W1, gin_b1,
           gin_bn_g, gin_bn_b, gin_W2, gin_b2, bn_g, bn_b, vn_W1, vn_b1,
           vn_bn1_g, vn_bn1_b, vn_W2, vn_b2, vn_bn2_g, vn_bn2_b):
    r2 = lambda a: a.reshape(1, D)

    pad = E_PAD - E
    src = jnp.concatenate([edge_index[0], jnp.zeros((pad,), jnp.int32)])
    dst = jnp.concatenate([edge_index[1], jnp.full((pad,), N, jnp.int32)])

    def pack(a):
        return a.reshape(NS, NQ, QC0, CH)

    idx0 = jnp.concatenate([pack(src), pack(dst)], axis=2)

    vn0 = vn_emb.astype(jnp.float32)

    def gin_weights(l):
        return (gin_W1[l], r2(gin_b1[l]), r2(gin_bn_g[l]), r2(gin_bn_b[l]),
                gin_W2[l], r2(gin_b2[l]), r2(bn_g[l]), r2(bn_b[l]))

    hvn0, r0, pooled0 = _k_in(x, W_in, r2(b_in), vn0)
    p = _sc_scatter(r0, idx0)
    vn1 = _k_vnm(pooled0, vn0, vn_W1[0], vn_b1[0], vn_bn1_g[0], vn_bn1_b[0],
                 vn_W2[0], vn_b2[0], vn_bn2_g[0], vn_bn2_b[0])
    hvn1, r1, nrep01, pooled1 = _k_mid(
        gin_eps[0].reshape(1), hvn0, p, gin_weights(0), vn1)

    p = _sc_scatter(r1, idx0)
    vn2 = _k_vnm(pooled1, vn1, vn_W1[1], vn_b1[1], vn_bn1_g[1], vn_bn1_b[1],
                 vn_W2[1], vn_b2[1], vn_bn2_g[1], vn_bn2_b[1])
    node_rep = _k_fin(
        gin_eps[1].reshape(1), hvn1, p, gin_weights(1), vn2, nrep01)
    return node_rep


# speedup vs baseline: 3.9447x; 3.9447x over previous
import functools

import jax
import jax.numpy as jnp
import numpy as np
from jax import lax
from jax.experimental import pallas as pl
from jax.experimental.pallas import tpu as pltpu
from jax.experimental.pallas import tpu_sc as plsc

N = 10000
D = 128
E = 320000
NC = 2
NS = 16
CH = 128
E_PAD = 327680
N_SP = 10112
ZROWS = N_SP // NS
OROWS = N_SP // NS


NQ = 4
KC = 80
QC0 = KC // NQ


def _sc_body(r_hbm, idx0_hbm, out_hbm,
             i0, i1, rows0, rows1, agg_sh, sem_i, sem_r0, sem_r1):
    c = lax.axis_index("c")
    s = lax.axis_index("s")
    t = c * NS + s

    def init():
        def zrow(i, carry):
            for j in range(8):
                rows0[i, pl.ds(16 * j, 16)] = jnp.zeros((16,), jnp.float32)
            return carry

        lax.fori_loop(0, CH, zrow, 0)
        zbase = s * ZROWS
        for k in range(ZROWS // CH):
            pltpu.sync_copy(rows0, agg_sh.at[pl.ds(zbase + k * CH, CH)])
        rem = ZROWS % CH
        pltpu.sync_copy(rows0.at[pl.ds(0, rem)],
                        agg_sh.at[pl.ds(zbase + ZROWS - rem, rem)])

    def run(idx_hbm, qch):
        pltpu.sync_copy(idx_hbm.at[t, 0], i0.at[pl.ds(0, 2 * qch)])
        plsc.subcore_barrier()

        def gather(iq, k, rows, sem):
            pltpu.async_copy(r_hbm.at[iq.at[k]], rows, sem)

        def gwait(iq, k, rows, sem):
            pltpu.make_async_copy(r_hbm.at[iq.at[k]], rows, sem).wait()

        def scat(iq, k, rows):
            pltpu.sync_copy(rows, agg_sh.at[iq.at[qch + k]], add=True)

        for q in range(NQ):
            iq, inx = (i0, i1) if q % 2 == 0 else (i1, i0)
            if q < NQ - 1:
                pltpu.async_copy(idx_hbm.at[t, q + 1],
                                 inx.at[pl.ds(0, 2 * qch)], sem_i)
            gather(iq, 0, rows0, sem_r0)

            def body(j, carry):
                gather(iq, 2 * j + 1, rows1, sem_r1)
                gwait(iq, 2 * j, rows0, sem_r0)
                scat(iq, 2 * j, rows0)
                gather(iq, 2 * j + 2, rows0, sem_r0)
                gwait(iq, 2 * j + 1, rows1, sem_r1)
                scat(iq, 2 * j + 1, rows1)
                return carry

            lax.fori_loop(0, qch // 2 - 1, body, 0)
            gather(iq, qch - 1, rows1, sem_r1)
            gwait(iq, qch - 2, rows0, sem_r0)
            scat(iq, qch - 2, rows0)
            gwait(iq, qch - 1, rows1, sem_r1)
            scat(iq, qch - 1, rows1)
            if q < NQ - 1:
                pltpu.make_async_copy(idx_hbm.at[t, q + 1],
                                      inx.at[pl.ds(0, 2 * qch)], sem_i).wait()

    init()
    run(idx0_hbm, QC0)
    plsc.subcore_barrier()
    pltpu.sync_copy(agg_sh.at[pl.ds(s * OROWS, OROWS)],
                    out_hbm.at[c, pl.ds(s * OROWS, OROWS)])


@functools.cache
def _make_sc_scatter():
    return pl.kernel(
        _sc_body,
        out_type=jax.ShapeDtypeStruct((NC, N_SP, D), jnp.float32),
        mesh=plsc.VectorSubcoreMesh(core_axis_name="c", subcore_axis_name="s",
                                    num_cores=NC, num_subcores=NS),
        scratch_types=[
            pltpu.VMEM((2 * QC0, CH), jnp.int32),
            pltpu.VMEM((2 * QC0, CH), jnp.int32),
            pltpu.VMEM((CH, D), jnp.float32),
            pltpu.VMEM((CH, D), jnp.float32),
            pltpu.VMEM_SHARED((N_SP, D), jnp.float32),
            pltpu.SemaphoreType.DMA,
            pltpu.SemaphoreType.DMA,
            pltpu.SemaphoreType.DMA,
        ],
    )


def _sc_scatter(r, idx0):
    return _make_sc_scatter()(r, idx0)


R = 1000
GRID = N // R

_CBN = float(np.float32(1.0) / np.sqrt(np.float32(1.0 + 1e-5)))


def _dot(a, b):
    return jnp.dot(a, b, preferred_element_type=jnp.float32)


def _kin_body(x_ref, w_ref, b_ref, vn_ref, hvn_ref, r_ref, pooled_ref):
    h0 = _dot(x_ref[...], w_ref[...]) + b_ref[...]
    hvn = h0 + vn_ref[...]
    hvn_ref[...] = hvn
    r_ref[...] = jnp.maximum(hvn, 0.0)
    ps = jnp.sum(hvn, axis=0, keepdims=True)

    @pl.when(pl.program_id(0) == 0)
    def _():
        pooled_ref[...] = ps

    @pl.when(pl.program_id(0) != 0)
    def _():
        pooled_ref[...] += ps


def _k_in(x, w_in, b_in, vn0):
    return pl.pallas_call(
        _kin_body,
        grid=(GRID,),
        in_specs=[
            pl.BlockSpec((R, D), lambda i: (i, 0)),
            pl.BlockSpec((D, D), lambda i: (0, 0)),
            pl.BlockSpec((1, D), lambda i: (0, 0)),
            pl.BlockSpec((1, D), lambda i: (0, 0)),
        ],
        out_specs=[
            pl.BlockSpec((R, D), lambda i: (i, 0)),
            pl.BlockSpec((R, D), lambda i: (i, 0)),
            pl.BlockSpec((1, D), lambda i: (0, 0)),
        ],
        out_shape=[
            jax.ShapeDtypeStruct((N, D), jnp.float32),
            jax.ShapeDtypeStruct((N, D), jnp.float32),
            jax.ShapeDtypeStruct((1, D), jnp.float32),
        ],
    )(x, w_in, b_in, vn0)


def _vnm_body(pooled_ref, vn_ref, w1_ref, b1_ref, g1_ref, be1_ref,
              w2_ref, b2_ref, g2_ref, be2_ref, out_ref):
    t = pooled_ref[...] + vn_ref[...]
    t = _dot(t, w1_ref[...]) + b1_ref[...]
    t = t * _CBN * g1_ref[...] + be1_ref[...]
    t = jnp.maximum(t, 0.0)
    t = _dot(t, w2_ref[...]) + b2_ref[...]
    t = t * _CBN * g2_ref[...] + be2_ref[...]
    out_ref[...] = jnp.maximum(t, 0.0)


def _k_vnm(pooled, vn, w1, b1, g1, be1, w2, b2, g2, be2):
    r2 = lambda a: a.reshape(1, D)
    return pl.pallas_call(
        _vnm_body,
        out_shape=jax.ShapeDtypeStruct((1, D), jnp.float32),
    )(pooled, vn, w1, r2(b1), r2(g1), r2(be1), w2, r2(b2), r2(g2), r2(be2))


def _gin_mlp(zsum, w1_ref, b1_ref, g1_ref, be1_ref, w2_ref, b2_ref,
             g2_ref, be2_ref):
    z = _dot(zsum, w1_ref[...]) + b1_ref[...]
    z = z * _CBN * g1_ref[...] + be1_ref[...]
    z = jnp.maximum(z, 0.0)
    z = _dot(z, w2_ref[...]) + b2_ref[...]
    z = z * _CBN * g2_ref[...] + be2_ref[...]
    return jnp.maximum(z, 0.0)


_W_SPECS = [
    pl.BlockSpec((D, D), lambda i: (0, 0)),
    pl.BlockSpec((1, D), lambda i: (0, 0)),
    pl.BlockSpec((1, D), lambda i: (0, 0)),
    pl.BlockSpec((1, D), lambda i: (0, 0)),
    pl.BlockSpec((D, D), lambda i: (0, 0)),
    pl.BlockSpec((1, D), lambda i: (0, 0)),
    pl.BlockSpec((1, D), lambda i: (0, 0)),
    pl.BlockSpec((1, D), lambda i: (0, 0)),
]


def _kmid_body(eps_ref, hvn_ref, p0_ref, p1_ref, w1_ref, b1_ref, g1_ref,
               be1_ref, w2_ref, b2_ref, g2_ref, be2_ref, vnn_ref,
               hvn1_ref, r1_ref, nrep_ref, pooled_ref):
    hvn = hvn_ref[...]
    zsum = (1.0 + eps_ref[0]) * hvn + (p0_ref[0] + p1_ref[0])
    out = _gin_mlp(zsum, w1_ref, b1_ref, g1_ref, be1_ref,
                   w2_ref, b2_ref, g2_ref, be2_ref)
    hvn1 = out + vnn_ref[...]
    hvn1_ref[...] = hvn1
    r1_ref[...] = jnp.maximum(hvn1, 0.0)
    nrep_ref[...] = hvn + hvn1
    ps = jnp.sum(hvn1, axis=0, keepdims=True)

    @pl.when(pl.program_id(0) == 0)
    def _():
        pooled_ref[...] = ps

    @pl.when(pl.program_id(0) != 0)
    def _():
        pooled_ref[...] += ps


def _k_mid(eps, hvn, p, weights, vnn):
    return pl.pallas_call(
        _kmid_body,
        grid=(GRID,),
        in_specs=[
            pl.BlockSpec(memory_space=pltpu.SMEM),
            pl.BlockSpec((R, D), lambda i: (i, 0)),
            pl.BlockSpec((1, R, D), lambda i: (0, i, 0)),
            pl.BlockSpec((1, R, D), lambda i: (1, i, 0)),
            *_W_SPECS,
            pl.BlockSpec((1, D), lambda i: (0, 0)),
        ],
        out_specs=[
            pl.BlockSpec((R, D), lambda i: (i, 0)),
            pl.BlockSpec((R, D), lambda i: (i, 0)),
            pl.BlockSpec((R, D), lambda i: (i, 0)),
            pl.BlockSpec((1, D), lambda i: (0, 0)),
        ],
        out_shape=[
            jax.ShapeDtypeStruct((N, D), jnp.float32),
            jax.ShapeDtypeStruct((N, D), jnp.float32),
            jax.ShapeDtypeStruct((N, D), jnp.float32),
            jax.ShapeDtypeStruct((1, D), jnp.float32),
        ],
    )(eps, hvn, p, p, *weights, vnn)


def _kfin_body(eps_ref, hvn_ref, p0_ref, p1_ref, w1_ref, b1_ref, g1_ref,
               be1_ref, w2_ref, b2_ref, g2_ref, be2_ref, vnn_ref, nrep_ref,
               out_ref):
    hvn = hvn_ref[...]
    zsum = (1.0 + eps_ref[0]) * hvn + (p0_ref[0] + p1_ref[0])
    out = _gin_mlp(zsum, w1_ref, b1_ref, g1_ref, be1_ref,
                   w2_ref, b2_ref, g2_ref, be2_ref)
    out_ref[...] = nrep_ref[...] + (out + vnn_ref[...])


def _k_fin(eps, hvn, p, weights, vnn, nrep):
    return pl.pallas_call(
        _kfin_body,
        grid=(GRID,),
        in_specs=[
            pl.BlockSpec(memory_space=pltpu.SMEM),
            pl.BlockSpec((R, D), lambda i: (i, 0)),
            pl.BlockSpec((1, R, D), lambda i: (0, i, 0)),
            pl.BlockSpec((1, R, D), lambda i: (1, i, 0)),
            *_W_SPECS,
            pl.BlockSpec((1, D), lambda i: (0, 0)),
            pl.BlockSpec((R, D), lambda i: (i, 0)),
        ],
        out_specs=pl.BlockSpec((R, D), lambda i: (i, 0)),
        out_shape=jax.ShapeDtypeStruct((N, D), jnp.float32),
    )(eps, hvn, p, p, *weights, vnn, nrep)


def kernel(x, edge_index, W_in, b_in, vn_emb, gin_eps, gin_W1, gin_b1,
           gin_bn_g, gin_bn_b, gin_W2, gin_b2, bn_g, bn_b, vn_W1, vn_b1,
           vn_bn1_g, vn_bn1_b, vn_W2, vn_b2, vn_bn2_g, vn_bn2_b):
    r2 = lambda a: a.reshape(1, D)

    pad = E_PAD - E
    pad_src = (jnp.arange(pad, dtype=jnp.int32) * 37) % N
    pad_dst = N + (jnp.arange(pad, dtype=jnp.int32) % (N_SP - N))
    src = jnp.concatenate([edge_index[0], pad_src])
    dst = jnp.concatenate([edge_index[1], pad_dst])

    def pack(a):
        return a.reshape(NC * NS, NQ, QC0, CH)

    idx0 = jnp.concatenate([pack(src), pack(dst)], axis=2)

    vn0 = vn_emb.astype(jnp.float32)

    def gin_weights(l):
        return (gin_W1[l], r2(gin_b1[l]), r2(gin_bn_g[l]), r2(gin_bn_b[l]),
                gin_W2[l], r2(gin_b2[l]), r2(bn_g[l]), r2(bn_b[l]))

    hvn0, r0, pooled0 = _k_in(x, W_in, r2(b_in), vn0)
    p = _sc_scatter(r0, idx0)
    vn1 = _k_vnm(pooled0, vn0, vn_W1[0], vn_b1[0], vn_bn1_g[0], vn_bn1_b[0],
                 vn_W2[0], vn_b2[0], vn_bn2_g[0], vn_bn2_b[0])
    hvn1, r1, nrep01, pooled1 = _k_mid(
        gin_eps[0].reshape(1), hvn0, p, gin_weights(0), vn1)

    p = _sc_scatter(r1, idx0)
    vn2 = _k_vnm(pooled1, vn1, vn_W1[1], vn_b1[1], vn_bn1_g[1], vn_bn1_b[1],
                 vn_W2[1], vn_b2[1], vn_bn2_g[1], vn_bn2_b[1])
    node_rep = _k_fin(
        gin_eps[1].reshape(1), hvn1, p, gin_weights(1), vn2, nrep01)
    return node_rep


# speedup vs baseline: 3.9448x; 1.0000x over previous
import functools

import jax
import jax.numpy as jnp
import numpy as np
from jax import lax
from jax.experimental import pallas as pl
from jax.experimental.pallas import tpu as pltpu
from jax.experimental.pallas import tpu_sc as plsc

N = 10000
D = 128
E = 320000
NC = 2
NS = 16
CH = 128
E_PAD = 327680
N_SP = 10112
ZROWS = N_SP // NS
OROWS = N_SP // NS


NQ = 4
KC = 80
QC0 = KC // NQ


def _sc_body(r_hbm, srcp_hbm, dstp_hbm, out_hbm,
             i0, i1, rows0, rows1, agg_sh, sem_i, sem_r0, sem_r1):
    c = lax.axis_index("c")
    s = lax.axis_index("s")
    t = c * NS + s

    def init():
        def zrow(i, carry):
            for j in range(8):
                rows0[i, pl.ds(16 * j, 16)] = jnp.zeros((16,), jnp.float32)
            return carry

        lax.fori_loop(0, CH, zrow, 0)
        zbase = s * ZROWS
        for k in range(ZROWS // CH):
            pltpu.sync_copy(rows0, agg_sh.at[pl.ds(zbase + k * CH, CH)])
        rem = ZROWS % CH
        pltpu.sync_copy(rows0.at[pl.ds(0, rem)],
                        agg_sh.at[pl.ds(zbase + ZROWS - rem, rem)])

    def run(qch):
        pltpu.sync_copy(srcp_hbm.at[t, 0], i0.at[0])
        pltpu.sync_copy(dstp_hbm.at[t, 0], i0.at[1])
        plsc.subcore_barrier()

        def gather(iq, k, rows, sem):
            pltpu.async_copy(r_hbm.at[iq.at[0, k]], rows, sem)

        def gwait(iq, k, rows, sem):
            pltpu.make_async_copy(r_hbm.at[iq.at[0, k]], rows, sem).wait()

        def scat(iq, k, rows):
            pltpu.sync_copy(rows, agg_sh.at[iq.at[1, k]], add=True)

        for q in range(NQ):
            iq, inx = (i0, i1) if q % 2 == 0 else (i1, i0)
            if q < NQ - 1:
                pltpu.async_copy(srcp_hbm.at[t, q + 1], inx.at[0], sem_i)
                pltpu.async_copy(dstp_hbm.at[t, q + 1], inx.at[1], sem_i)
            gather(iq, 0, rows0, sem_r0)

            def body(j, carry):
                gather(iq, 2 * j + 1, rows1, sem_r1)
                gwait(iq, 2 * j, rows0, sem_r0)
                scat(iq, 2 * j, rows0)
                gather(iq, 2 * j + 2, rows0, sem_r0)
                gwait(iq, 2 * j + 1, rows1, sem_r1)
                scat(iq, 2 * j + 1, rows1)
                return carry

            lax.fori_loop(0, qch // 2 - 1, body, 0)
            gather(iq, qch - 1, rows1, sem_r1)
            gwait(iq, qch - 2, rows0, sem_r0)
            scat(iq, qch - 2, rows0)
            gwait(iq, qch - 1, rows1, sem_r1)
            scat(iq, qch - 1, rows1)
            if q < NQ - 1:
                pltpu.make_async_copy(srcp_hbm.at[t, q + 1], inx.at[0],
                                      sem_i).wait()
                pltpu.make_async_copy(dstp_hbm.at[t, q + 1], inx.at[1],
                                      sem_i).wait()

    init()
    run(QC0)
    plsc.subcore_barrier()
    pltpu.sync_copy(agg_sh.at[pl.ds(s * OROWS, OROWS)],
                    out_hbm.at[c, pl.ds(s * OROWS, OROWS)])


@functools.cache
def _make_sc_scatter():
    return pl.kernel(
        _sc_body,
        out_type=jax.ShapeDtypeStruct((NC, N_SP, D), jnp.float32),
        mesh=plsc.VectorSubcoreMesh(core_axis_name="c", subcore_axis_name="s",
                                    num_cores=NC, num_subcores=NS),
        scratch_types=[
            pltpu.VMEM((2, QC0, CH), jnp.int32),
            pltpu.VMEM((2, QC0, CH), jnp.int32),
            pltpu.VMEM((CH, D), jnp.float32),
            pltpu.VMEM((CH, D), jnp.float32),
            pltpu.VMEM_SHARED((N_SP, D), jnp.float32),
            pltpu.SemaphoreType.DMA,
            pltpu.SemaphoreType.DMA,
            pltpu.SemaphoreType.DMA,
        ],
    )


def _sc_scatter(r, srcp, dstp):
    return _make_sc_scatter()(r, srcp, dstp)


R = 1000
GRID = N // R

_CBN = float(np.float32(1.0) / np.sqrt(np.float32(1.0 + 1e-5)))


def _dot(a, b):
    return jnp.dot(a, b, preferred_element_type=jnp.float32)


def _kin_body(x_ref, w_ref, b_ref, vn_ref, hvn_ref, r_ref, pooled_ref):
    h0 = _dot(x_ref[...], w_ref[...]) + b_ref[...]
    hvn = h0 + vn_ref[...]
    hvn_ref[...] = hvn
    r_ref[...] = jnp.maximum(hvn, 0.0)
    ps = jnp.sum(hvn, axis=0, keepdims=True)

    @pl.when(pl.program_id(0) == 0)
    def _():
        pooled_ref[...] = ps

    @pl.when(pl.program_id(0) != 0)
    def _():
        pooled_ref[...] += ps


def _k_in(x, w_in, b_in, vn0):
    return pl.pallas_call(
        _kin_body,
        grid=(GRID,),
        in_specs=[
            pl.BlockSpec((R, D), lambda i: (i, 0)),
            pl.BlockSpec((D, D), lambda i: (0, 0)),
            pl.BlockSpec((1, D), lambda i: (0, 0)),
            pl.BlockSpec((1, D), lambda i: (0, 0)),
        ],
        out_specs=[
            pl.BlockSpec((R, D), lambda i: (i, 0)),
            pl.BlockSpec((R, D), lambda i: (i, 0)),
            pl.BlockSpec((1, D), lambda i: (0, 0)),
        ],
        out_shape=[
            jax.ShapeDtypeStruct((N, D), jnp.float32),
            jax.ShapeDtypeStruct((N, D), jnp.float32),
            jax.ShapeDtypeStruct((1, D), jnp.float32),
        ],
    )(x, w_in, b_in, vn0)


def _vnm_body(pooled_ref, vn_ref, w1_ref, b1_ref, g1_ref, be1_ref,
              w2_ref, b2_ref, g2_ref, be2_ref, out_ref):
    t = pooled_ref[...] + vn_ref[...]
    t = _dot(t, w1_ref[...]) + b1_ref[...]
    t = t * _CBN * g1_ref[...] + be1_ref[...]
    t = jnp.maximum(t, 0.0)
    t = _dot(t, w2_ref[...]) + b2_ref[...]
    t = t * _CBN * g2_ref[...] + be2_ref[...]
    out_ref[...] = jnp.maximum(t, 0.0)


def _k_vnm(pooled, vn, w1, b1, g1, be1, w2, b2, g2, be2):
    r2 = lambda a: a.reshape(1, D)
    return pl.pallas_call(
        _vnm_body,
        out_shape=jax.ShapeDtypeStruct((1, D), jnp.float32),
    )(pooled, vn, w1, r2(b1), r2(g1), r2(be1), w2, r2(b2), r2(g2), r2(be2))


def _gin_mlp(zsum, w1_ref, b1_ref, g1_ref, be1_ref, w2_ref, b2_ref,
             g2_ref, be2_ref):
    z = _dot(zsum, w1_ref[...]) + b1_ref[...]
    z = z * _CBN * g1_ref[...] + be1_ref[...]
    z = jnp.maximum(z, 0.0)
    z = _dot(z, w2_ref[...]) + b2_ref[...]
    z = z * _CBN * g2_ref[...] + be2_ref[...]
    return jnp.maximum(z, 0.0)


_W_SPECS = [
    pl.BlockSpec((D, D), lambda i: (0, 0)),
    pl.BlockSpec((1, D), lambda i: (0, 0)),
    pl.BlockSpec((1, D), lambda i: (0, 0)),
    pl.BlockSpec((1, D), lambda i: (0, 0)),
    pl.BlockSpec((D, D), lambda i: (0, 0)),
    pl.BlockSpec((1, D), lambda i: (0, 0)),
    pl.BlockSpec((1, D), lambda i: (0, 0)),
    pl.BlockSpec((1, D), lambda i: (0, 0)),
]


def _kmid_body(eps_ref, hvn_ref, p0_ref, p1_ref, w1_ref, b1_ref, g1_ref,
               be1_ref, w2_ref, b2_ref, g2_ref, be2_ref, vnn_ref,
               hvn1_ref, r1_ref, nrep_ref, pooled_ref):
    hvn = hvn_ref[...]
    zsum = (1.0 + eps_ref[0]) * hvn + (p0_ref[0] + p1_ref[0])
    out = _gin_mlp(zsum, w1_ref, b1_ref, g1_ref, be1_ref,
                   w2_ref, b2_ref, g2_ref, be2_ref)
    hvn1 = out + vnn_ref[...]
    hvn1_ref[...] = hvn1
    r1_ref[...] = jnp.maximum(hvn1, 0.0)
    nrep_ref[...] = hvn + hvn1
    ps = jnp.sum(hvn1, axis=0, keepdims=True)

    @pl.when(pl.program_id(0) == 0)
    def _():
        pooled_ref[...] = ps

    @pl.when(pl.program_id(0) != 0)
    def _():
        pooled_ref[...] += ps


def _k_mid(eps, hvn, p, weights, vnn):
    return pl.pallas_call(
        _kmid_body,
        grid=(GRID,),
        in_specs=[
            pl.BlockSpec(memory_space=pltpu.SMEM),
            pl.BlockSpec((R, D), lambda i: (i, 0)),
            pl.BlockSpec((1, R, D), lambda i: (0, i, 0)),
            pl.BlockSpec((1, R, D), lambda i: (1, i, 0)),
            *_W_SPECS,
            pl.BlockSpec((1, D), lambda i: (0, 0)),
        ],
        out_specs=[
            pl.BlockSpec((R, D), lambda i: (i, 0)),
            pl.BlockSpec((R, D), lambda i: (i, 0)),
            pl.BlockSpec((R, D), lambda i: (i, 0)),
            pl.BlockSpec((1, D), lambda i: (0, 0)),
        ],
        out_shape=[
            jax.ShapeDtypeStruct((N, D), jnp.float32),
            jax.ShapeDtypeStruct((N, D), jnp.float32),
            jax.ShapeDtypeStruct((N, D), jnp.float32),
            jax.ShapeDtypeStruct((1, D), jnp.float32),
        ],
    )(eps, hvn, p, p, *weights, vnn)


def _kfin_body(eps_ref, hvn_ref, p0_ref, p1_ref, w1_ref, b1_ref, g1_ref,
               be1_ref, w2_ref, b2_ref, g2_ref, be2_ref, vnn_ref, nrep_ref,
               out_ref):
    hvn = hvn_ref[...]
    zsum = (1.0 + eps_ref[0]) * hvn + (p0_ref[0] + p1_ref[0])
    out = _gin_mlp(zsum, w1_ref, b1_ref, g1_ref, be1_ref,
                   w2_ref, b2_ref, g2_ref, be2_ref)
    out_ref[...] = nrep_ref[...] + (out + vnn_ref[...])


def _k_fin(eps, hvn, p, weights, vnn, nrep):
    return pl.pallas_call(
        _kfin_body,
        grid=(GRID,),
        in_specs=[
            pl.BlockSpec(memory_space=pltpu.SMEM),
            pl.BlockSpec((R, D), lambda i: (i, 0)),
            pl.BlockSpec((1, R, D), lambda i: (0, i, 0)),
            pl.BlockSpec((1, R, D), lambda i: (1, i, 0)),
            *_W_SPECS,
            pl.BlockSpec((1, D), lambda i: (0, 0)),
            pl.BlockSpec((R, D), lambda i: (i, 0)),
        ],
        out_specs=pl.BlockSpec((R, D), lambda i: (i, 0)),
        out_shape=jax.ShapeDtypeStruct((N, D), jnp.float32),
    )(eps, hvn, p, p, *weights, vnn, nrep)


def kernel(x, edge_index, W_in, b_in, vn_emb, gin_eps, gin_W1, gin_b1,
           gin_bn_g, gin_bn_b, gin_W2, gin_b2, bn_g, bn_b, vn_W1, vn_b1,
           vn_bn1_g, vn_bn1_b, vn_W2, vn_b2, vn_bn2_g, vn_bn2_b):
    r2 = lambda a: a.reshape(1, D)

    pad = E_PAD - E
    pad_src = (jnp.arange(pad, dtype=jnp.int32) * 37) % N
    pad_dst = N + (jnp.arange(pad, dtype=jnp.int32) % (N_SP - N))
    src = jnp.concatenate([edge_index[0], pad_src])
    dst = jnp.concatenate([edge_index[1], pad_dst])

    srcp = src.reshape(NC * NS, NQ, QC0, CH)
    dstp = dst.reshape(NC * NS, NQ, QC0, CH)

    vn0 = vn_emb.astype(jnp.float32)

    def gin_weights(l):
        return (gin_W1[l], r2(gin_b1[l]), r2(gin_bn_g[l]), r2(gin_bn_b[l]),
                gin_W2[l], r2(gin_b2[l]), r2(bn_g[l]), r2(bn_b[l]))

    hvn0, r0, pooled0 = _k_in(x, W_in, r2(b_in), vn0)
    p = _sc_scatter(r0, srcp, dstp)
    vn1 = _k_vnm(pooled0, vn0, vn_W1[0], vn_b1[0], vn_bn1_g[0], vn_bn1_b[0],
                 vn_W2[0], vn_b2[0], vn_bn2_g[0], vn_bn2_b[0])
    hvn1, r1, nrep01, pooled1 = _k_mid(
        gin_eps[0].reshape(1), hvn0, p, gin_weights(0), vn1)

    p = _sc_scatter(r1, srcp, dstp)
    vn2 = _k_vnm(pooled1, vn1, vn_W1[1], vn_b1[1], vn_bn1_g[1], vn_bn1_b[1],
                 vn_W2[1], vn_b2[1], vn_bn2_g[1], vn_bn2_b[1])
    node_rep = _k_fin(
        gin_eps[1].reshape(1), hvn1, p, gin_weights(1), vn2, nrep01)
    return node_rep
